# Initial kernel scaffold; baseline (speedup 1.0000x reference)
import jax
import jax.numpy as jnp
from jax.experimental import pallas as pl


def kernel(x, simi_mat, conv_w, conv_b, wq, bq, wk, bk, wv, bv, wout, bout):
    raise NotImplementedError("write your pallas kernel here")


# speedup vs baseline: 1.5508x; 1.5508x over previous
import jax
import jax.numpy as jnp
import numpy as np
from jax.experimental import pallas as pl

_RB = 256
_TOPK = 32


def _stage_a(pb_ref, wc_ref, cb_ref, wq_ref, bq_ref, wk_ref, bk_ref,
             wv_ref, bv_ref, q_ref, k_ref, v_ref):
    pb = pb_ref[0]
    f = jnp.dot(pb, wc_ref[...], preferred_element_type=jnp.float32)
    f = f + cb_ref[...]
    f = jnp.where(f >= 0, f, 0.01 * f)
    q_ref[0] = jnp.dot(f, wq_ref[...], preferred_element_type=jnp.float32) + bq_ref[...]
    k_ref[0] = jnp.dot(f, wk_ref[...], preferred_element_type=jnp.float32) + bk_ref[...]
    v_ref[0] = jnp.dot(f, wv_ref[...], preferred_element_type=jnp.float32) + bv_ref[...]


def _make_stage_b(n, rb, sw, scale):
    def _stage_b(q_ref, k_ref, v_ref, st_ref, o_ref):
        q = q_ref[0]
        k = k_ref[0]
        st = st_ref[0]
        s = jax.lax.dot_general(q, k, (((1,), (1,)), ((), ())),
                                preferred_element_type=jnp.float32) * scale
        m = jnp.max(s, axis=1, keepdims=True)
        p = jnp.exp(s - m)
        p = p / jnp.sum(p, axis=1, keepdims=True)
        col_iota = jax.lax.broadcasted_iota(jnp.int32, (rb, n), 1)
        row_ids = pl.program_id(1) * rb + jax.lax.broadcasted_iota(
            jnp.int32, (rb, 1), 0)
        factor = jnp.ones((rb, n), jnp.float32)
        for t in range(_TOPK):
            col = st[t, :].reshape(rb, 1)
            col = jnp.where(col == -1, row_ids, col)
            factor = jnp.where(col == col_iota, sw[t], factor)
        p = p * factor
        o_ref[0] = jnp.dot(p, v_ref[0], preferred_element_type=jnp.float32)
    return _stage_b


def _stage_c(ao_ref, wo_ref, bo_ref, out_ref):
    ao = ao_ref[0]
    o = jnp.dot(wo_ref[...], ao, preferred_element_type=jnp.float32)
    o = o + bo_ref[...]
    om = jnp.max(o, axis=0, keepdims=True)
    e = jnp.exp(o - om)
    out_ref[0] = e / jnp.sum(e, axis=0, keepdims=True)


def kernel(x, simi_mat, conv_w, conv_b, wq, bq, wk, bk, wv, bv, wout, bout):
    b, cin, hh, ww = x.shape
    cout = conv_w.shape[0]
    p = conv_w.shape[2]
    nh, nw = hh // p, ww // p
    n = nh * nw
    feat = cin * p * p

    patches = x.reshape(b, cin, nh, p, nw, p).transpose(0, 2, 4, 1, 3, 5)
    patches = patches.reshape(b, n, feat)
    wc = conv_w.reshape(cout, feat).T
    simi_t = simi_mat.transpose(0, 2, 1)

    cb2 = conv_b.reshape(1, cout)
    bq2 = bq.reshape(1, cout)
    bk2 = bk.reshape(1, cout)
    bv2 = bv.reshape(1, cout)
    bo2 = bout.reshape(1, 1)

    nblk = n // _RB
    grid_ab = (b, nblk)

    q, k, v = pl.pallas_call(
        _stage_a,
        grid=grid_ab,
        in_specs=[
            pl.BlockSpec((1, _RB, feat), lambda i, j: (i, j, 0)),
            pl.BlockSpec((feat, cout), lambda i, j: (0, 0)),
            pl.BlockSpec((1, cout), lambda i, j: (0, 0)),
            pl.BlockSpec((cout, cout), lambda i, j: (0, 0)),
            pl.BlockSpec((1, cout), lambda i, j: (0, 0)),
            pl.BlockSpec((cout, cout), lambda i, j: (0, 0)),
            pl.BlockSpec((1, cout), lambda i, j: (0, 0)),
            pl.BlockSpec((cout, cout), lambda i, j: (0, 0)),
            pl.BlockSpec((1, cout), lambda i, j: (0, 0)),
        ],
        out_specs=[
            pl.BlockSpec((1, _RB, cout), lambda i, j: (i, j, 0)),
            pl.BlockSpec((1, _RB, cout), lambda i, j: (i, j, 0)),
            pl.BlockSpec((1, _RB, cout), lambda i, j: (i, j, 0)),
        ],
        out_shape=[jax.ShapeDtypeStruct((b, n, cout), jnp.float32)] * 3,
    )(patches, wc, cb2, wq.T, bq2, wk.T, bk2, wv.T, bv2)

    hk = np.reciprocal(np.arange(1, _TOPK + 1, dtype=np.float64))
    sw = tuple((hk / hk.sum()).astype(np.float32).tolist())
    scale = float(1.0 / np.sqrt(np.float32(cout)))

    attn_out = pl.pallas_call(
        _make_stage_b(n, _RB, sw, scale),
        grid=grid_ab,
        in_specs=[
            pl.BlockSpec((1, _RB, cout), lambda i, j: (i, j, 0)),
            pl.BlockSpec((1, n, cout), lambda i, j: (i, 0, 0)),
            pl.BlockSpec((1, n, cout), lambda i, j: (i, 0, 0)),
            pl.BlockSpec((1, _TOPK, _RB), lambda i, j: (i, 0, j)),
        ],
        out_specs=pl.BlockSpec((1, _RB, cout), lambda i, j: (i, j, 0)),
        out_shape=jax.ShapeDtypeStruct((b, n, cout), jnp.float32),
    )(q, k, v, simi_t)

    out = pl.pallas_call(
        _stage_c,
        grid=(b,),
        in_specs=[
            pl.BlockSpec((1, n, cout), lambda i: (i, 0, 0)),
            pl.BlockSpec((1, n), lambda i: (0, 0)),
            pl.BlockSpec((1, 1), lambda i: (0, 0)),
        ],
        out_specs=pl.BlockSpec((1, 1, cout), lambda i: (i, 0, 0)),
        out_shape=jax.ShapeDtypeStruct((b, 1, cout), jnp.float32),
    )(attn_out, wout, bo2)

    return out.reshape(b, cout, 1)


# speedup vs baseline: 7.1394x; 4.6036x over previous
import jax
import jax.numpy as jnp
import numpy as np
from jax.experimental import pallas as pl

_RB = 256
_TOPK = 32
_P = 8


def _stage_a(x_ref, wc_ref, cb_ref, wq_ref, bq_ref, wk_ref, bk_ref,
             wv_ref, bv_ref, q_ref, k_ref, v_ref):
    xb = x_ref[0]
    cin, hb, w = xb.shape
    pieces = []
    for phs in range(hb // _P):
        sub = xb[:, phs * _P:(phs + 1) * _P, :]
        sub2 = sub.reshape(cin * _P, w)
        t = sub2.T
        pieces.append(t.reshape(w // _P, _P * cin * _P))
    patches = jnp.concatenate(pieces, axis=0)
    f = jnp.dot(patches, wc_ref[...], preferred_element_type=jnp.float32)
    f = f + cb_ref[...]
    f = jnp.where(f >= 0, f, 0.01 * f)
    q_ref[0] = jnp.dot(f, wq_ref[...], preferred_element_type=jnp.float32) + bq_ref[...]
    k_ref[0] = jnp.dot(f, wk_ref[...], preferred_element_type=jnp.float32) + bk_ref[...]
    v_ref[0] = jnp.dot(f, wv_ref[...], preferred_element_type=jnp.float32) + bv_ref[...]


def _make_stage_b(n, rb, nblk, sw, scale):
    def _stage_b(q_ref, k_ref, v_ref, st_ref, wo_ref, bo_ref, o_ref):
        jb = pl.program_id(1)
        q = q_ref[0]
        k = k_ref[0]
        st = st_ref[0]
        s = jax.lax.dot_general(q, k, (((1,), (1,)), ((), ())),
                                preferred_element_type=jnp.float32) * scale
        m = jnp.max(s, axis=1, keepdims=True)
        p = jnp.exp(s - m)
        p = p / jnp.sum(p, axis=1, keepdims=True)
        col_iota = jax.lax.broadcasted_iota(jnp.int32, (rb, n), 1)
        row_ids = jb * rb + jax.lax.broadcasted_iota(jnp.int32, (rb, 1), 0)
        factor = jnp.ones((rb, n), jnp.float32)
        for t in range(_TOPK):
            col = st[t, :].reshape(rb, 1)
            col = jnp.where(col == -1, row_ids, col)
            factor = jnp.where(col == col_iota, sw[t], factor)
        p = p * factor
        ao = jnp.dot(p, v_ref[0], preferred_element_type=jnp.float32)
        contrib = jnp.dot(wo_ref[...], ao, preferred_element_type=jnp.float32)

        @pl.when(jb == 0)
        def _init():
            o_ref[0] = contrib + bo_ref[...]

        @pl.when(jb > 0)
        def _acc():
            o_ref[0] += contrib

        @pl.when(jb == nblk - 1)
        def _fin():
            val = o_ref[0]
            e = jnp.exp(val - val)
            o_ref[0] = e / e

    return _stage_b


def kernel(x, simi_mat, conv_w, conv_b, wq, bq, wk, bk, wv, bv, wout, bout):
    b, cin, hh, ww = x.shape
    cout = conv_w.shape[0]
    nh, nw = hh // _P, ww // _P
    n = nh * nw
    feat = cin * _P * _P
    nblk = n // _RB
    hrows = (_RB // nw) * _P

    wc = conv_w.transpose(3, 1, 2, 0).reshape(feat, cout)
    simi_t = simi_mat.transpose(0, 2, 1)
    cb2 = conv_b.reshape(1, cout)
    bq2 = bq.reshape(1, cout)
    bk2 = bk.reshape(1, cout)
    bv2 = bv.reshape(1, cout)
    bo2 = bout.reshape(1, 1)

    grid = (b, nblk)

    q, k, v = pl.pallas_call(
        _stage_a,
        grid=grid,
        in_specs=[
            pl.BlockSpec((1, cin, hrows, ww), lambda i, j: (i, 0, j, 0)),
            pl.BlockSpec((feat, cout), lambda i, j: (0, 0)),
            pl.BlockSpec((1, cout), lambda i, j: (0, 0)),
            pl.BlockSpec((cout, cout), lambda i, j: (0, 0)),
            pl.BlockSpec((1, cout), lambda i, j: (0, 0)),
            pl.BlockSpec((cout, cout), lambda i, j: (0, 0)),
            pl.BlockSpec((1, cout), lambda i, j: (0, 0)),
            pl.BlockSpec((cout, cout), lambda i, j: (0, 0)),
            pl.BlockSpec((1, cout), lambda i, j: (0, 0)),
        ],
        out_specs=[
            pl.BlockSpec((1, _RB, cout), lambda i, j: (i, j, 0)),
            pl.BlockSpec((1, _RB, cout), lambda i, j: (i, j, 0)),
            pl.BlockSpec((1, _RB, cout), lambda i, j: (i, j, 0)),
        ],
        out_shape=[jax.ShapeDtypeStruct((b, n, cout), jnp.float32)] * 3,
    )(x, wc, cb2, wq.T, bq2, wk.T, bk2, wv.T, bv2)

    hk = np.reciprocal(np.arange(1, _TOPK + 1, dtype=np.float64))
    sw = tuple((hk / hk.sum()).astype(np.float32).tolist())
    scale = float(1.0 / np.sqrt(np.float32(cout)))

    out = pl.pallas_call(
        _make_stage_b(n, _RB, nblk, sw, scale),
        grid=grid,
        in_specs=[
            pl.BlockSpec((1, _RB, cout), lambda i, j: (i, j, 0)),
            pl.BlockSpec((1, n, cout), lambda i, j: (i, 0, 0)),
            pl.BlockSpec((1, n, cout), lambda i, j: (i, 0, 0)),
            pl.BlockSpec((1, _TOPK, _RB), lambda i, j: (i, 0, j)),
            pl.BlockSpec((1, _RB), lambda i, j: (0, j)),
            pl.BlockSpec((1, 1), lambda i, j: (0, 0)),
        ],
        out_specs=pl.BlockSpec((1, 1, cout), lambda i, j: (i, 0, 0)),
        out_shape=jax.ShapeDtypeStruct((b, 1, cout), jnp.float32),
    )(q, k, v, simi_t, wout, bo2)

    return out.reshape(b, cout, 1)


# speedup vs baseline: 8.1709x; 1.1445x over previous
import jax
import jax.numpy as jnp
import numpy as np
from jax.experimental import pallas as pl

_RB = 256
_TOPK = 32
_P = 8


def _stage_a(x_ref, wc_ref, cb_ref, wq_ref, bq_ref, wk_ref, bk_ref,
             wv_ref, bv_ref, q_ref, k_ref, v_ref):
    xb = x_ref[0]
    cin, hb, w = xb.shape
    nwb = w // _P
    pieces = []
    for phs in range(hb // _P):
        sub = xb[:, phs * _P:(phs + 1) * _P, :]
        sub2 = sub.reshape(cin * _P, w)
        t = sub2.T
        pieces.append(t.reshape(nwb, _P, cin * _P))
    t3 = jnp.concatenate(pieces, axis=0)
    f = jnp.zeros((t3.shape[0], cb_ref.shape[1]), jnp.float32) + cb_ref[...]
    for dw in range(_P):
        f = f + jnp.dot(t3[:, dw, :], wc_ref[dw],
                        preferred_element_type=jnp.float32)
    f = jnp.where(f >= 0, f, 0.01 * f)
    q_ref[0] = jnp.dot(f, wq_ref[...], preferred_element_type=jnp.float32) + bq_ref[...]
    k_ref[0] = jnp.dot(f, wk_ref[...], preferred_element_type=jnp.float32) + bk_ref[...]
    v_ref[0] = jnp.dot(f, wv_ref[...], preferred_element_type=jnp.float32) + bv_ref[...]


def _make_stage_b(n, rb, nblk, hsum, scale):
    def _stage_b(q_ref, k_ref, v_ref, st_ref, wo_ref, bo_ref, o_ref):
        jb = pl.program_id(1)
        q = q_ref[0]
        k = k_ref[0]
        s = jax.lax.dot_general(q, k, (((1,), (1,)), ((), ())),
                                preferred_element_type=jnp.float32) * scale
        m = jnp.max(s, axis=1, keepdims=True)
        p = jnp.exp(s - m)
        p = p / jnp.sum(p, axis=1, keepdims=True)
        row_ids = (jb * rb + jax.lax.broadcasted_iota(jnp.int32, (rb, 1), 0)
                   ).astype(jnp.int16)
        st = st_ref[0].astype(jnp.int16)
        st = jnp.where(st == -1, row_ids, st)
        col_iota = jax.lax.broadcasted_iota(jnp.int16, (rb, n), 1)
        win = jnp.full((rb, n), -1, jnp.int16)
        for t in range(_TOPK):
            col = st[:, t:t + 1]
            win = jnp.where(col == col_iota, jnp.int16(t), win)
        winf = win.astype(jnp.float32)
        factor = jnp.where(win >= 0, 1.0 / ((winf + 1.0) * hsum), 1.0)
        p = p * factor
        ao = jnp.dot(p, v_ref[0], preferred_element_type=jnp.float32)
        contrib = jnp.dot(wo_ref[...], ao, preferred_element_type=jnp.float32)

        @pl.when(jb == 0)
        def _init():
            o_ref[0] = contrib + bo_ref[...]

        @pl.when(jb > 0)
        def _acc():
            o_ref[0] += contrib

        @pl.when(jb == nblk - 1)
        def _fin():
            val = o_ref[0]
            e = jnp.exp(val - val)
            o_ref[0] = e / e

    return _stage_b


def kernel(x, simi_mat, conv_w, conv_b, wq, bq, wk, bk, wv, bv, wout, bout):
    b, cin, hh, ww = x.shape
    cout = conv_w.shape[0]
    nh, nw = hh // _P, ww // _P
    n = nh * nw
    nblk = n // _RB
    hrows = (_RB // nw) * _P

    wc = conv_w.transpose(3, 1, 2, 0).reshape(_P, cin * _P, cout)
    cb2 = conv_b.reshape(1, cout)
    bq2 = bq.reshape(1, cout)
    bk2 = bk.reshape(1, cout)
    bv2 = bv.reshape(1, cout)
    bo2 = bout.reshape(1, 1)

    grid = (b, nblk)

    q, k, v = pl.pallas_call(
        _stage_a,
        grid=grid,
        in_specs=[
            pl.BlockSpec((1, cin, hrows, ww), lambda i, j: (i, 0, j, 0)),
            pl.BlockSpec((_P, cin * _P, cout), lambda i, j: (0, 0, 0)),
            pl.BlockSpec((1, cout), lambda i, j: (0, 0)),
            pl.BlockSpec((cout, cout), lambda i, j: (0, 0)),
            pl.BlockSpec((1, cout), lambda i, j: (0, 0)),
            pl.BlockSpec((cout, cout), lambda i, j: (0, 0)),
            pl.BlockSpec((1, cout), lambda i, j: (0, 0)),
            pl.BlockSpec((cout, cout), lambda i, j: (0, 0)),
            pl.BlockSpec((1, cout), lambda i, j: (0, 0)),
        ],
        out_specs=[
            pl.BlockSpec((1, _RB, cout), lambda i, j: (i, j, 0)),
            pl.BlockSpec((1, _RB, cout), lambda i, j: (i, j, 0)),
            pl.BlockSpec((1, _RB, cout), lambda i, j: (i, j, 0)),
        ],
        out_shape=[jax.ShapeDtypeStruct((b, n, cout), jnp.float32)] * 3,
    )(x, wc, cb2, wq.T, bq2, wk.T, bk2, wv.T, bv2)

    hk = np.reciprocal(np.arange(1, _TOPK + 1, dtype=np.float64))
    hsum = float(np.float32(hk.sum()))
    scale = float(1.0 / np.sqrt(np.float32(cout)))

    out = pl.pallas_call(
        _make_stage_b(n, _RB, nblk, hsum, scale),
        grid=grid,
        in_specs=[
            pl.BlockSpec((1, _RB, cout), lambda i, j: (i, j, 0)),
            pl.BlockSpec((1, n, cout), lambda i, j: (i, 0, 0)),
            pl.BlockSpec((1, n, cout), lambda i, j: (i, 0, 0)),
            pl.BlockSpec((1, _RB, _TOPK), lambda i, j: (i, j, 0)),
            pl.BlockSpec((1, _RB), lambda i, j: (0, j)),
            pl.BlockSpec((1, 1), lambda i, j: (0, 0)),
        ],
        out_specs=pl.BlockSpec((1, 1, cout), lambda i, j: (i, 0, 0)),
        out_shape=jax.ShapeDtypeStruct((b, 1, cout), jnp.float32),
    )(q, k, v, simi_mat, wout, bo2)

    return out.reshape(b, cout, 1)


# speedup vs baseline: 8.7466x; 1.0705x over previous
import jax
import jax.numpy as jnp
import numpy as np
from jax.experimental import pallas as pl
from jax.experimental.pallas import tpu as pltpu

_RB = 256
_TOPK = 32
_P = 8


def _stage_a(x_ref, wc_ref, cb_ref, wq_ref, bq_ref, wk_ref, bk_ref,
             wv_ref, bv_ref, q_ref, k_ref, v_ref):
    xb = x_ref[0]
    cin, hb, w = xb.shape
    nwb = w // _P
    pieces = []
    for phs in range(hb // _P):
        sub = xb[:, phs * _P:(phs + 1) * _P, :]
        sub2 = sub.reshape(cin * _P, w)
        t = sub2.T
        pieces.append(t.reshape(nwb, _P * cin * _P))
    patches = jnp.concatenate(pieces, axis=0)
    f = jnp.dot(patches, wc_ref[...], preferred_element_type=jnp.float32)
    f = f + cb_ref[...]
    f = jnp.where(f >= 0, f, 0.01 * f)
    q_ref[0] = jnp.dot(f, wq_ref[...], preferred_element_type=jnp.float32) + bq_ref[...]
    k_ref[0] = jnp.dot(f, wk_ref[...], preferred_element_type=jnp.float32) + bk_ref[...]
    v_ref[0] = jnp.dot(f, wv_ref[...], preferred_element_type=jnp.float32) + bv_ref[...]


def _make_stage_b(n, rb, nblk, hsum, scale):
    def _stage_b(q_ref, k_ref, v_ref, st_ref, wo_ref, bo_ref, o_ref):
        jb = pl.program_id(1)
        q = q_ref[0]
        k = k_ref[0]
        s = jax.lax.dot_general(q, k, (((1,), (1,)), ((), ())),
                                preferred_element_type=jnp.float32) * scale
        m = jnp.max(s, axis=1, keepdims=True)
        p = jnp.exp(s - m)
        p = p / jnp.sum(p, axis=1, keepdims=True)
        row_ids = (jb * rb + jax.lax.broadcasted_iota(jnp.int32, (rb, 1), 0)
                   ).astype(jnp.int16)
        st = st_ref[0].astype(jnp.int16)
        st = jnp.where(st == -1, row_ids, st)
        col_iota = jax.lax.broadcasted_iota(jnp.int16, (rb, n), 1)
        win = jnp.full((rb, n), -1, jnp.int16)
        for t in range(_TOPK):
            col = st[:, t:t + 1]
            win = jnp.where(col == col_iota, jnp.int16(t), win)
        winf = win.astype(jnp.float32)
        factor = jnp.where(win >= 0, 1.0 / ((winf + 1.0) * hsum), 1.0)
        p = p * factor
        ao = jnp.dot(p, v_ref[0], preferred_element_type=jnp.float32)
        contrib = jnp.dot(wo_ref[...], ao, preferred_element_type=jnp.float32)

        @pl.when(jb == 0)
        def _init():
            o_ref[0] = contrib + bo_ref[...]

        @pl.when(jb > 0)
        def _acc():
            o_ref[0] += contrib

        @pl.when(jb == nblk - 1)
        def _fin():
            val = o_ref[0]
            e = jnp.exp(val - val)
            o_ref[0] = e / e

    return _stage_b


def kernel(x, simi_mat, conv_w, conv_b, wq, bq, wk, bk, wv, bv, wout, bout):
    b, cin, hh, ww = x.shape
    cout = conv_w.shape[0]
    nh, nw = hh // _P, ww // _P
    n = nh * nw
    nblk = n // _RB
    hrows = (_RB // nw) * _P

    wc = conv_w.transpose(3, 1, 2, 0).reshape(_P * cin * _P, cout)
    cb2 = conv_b.reshape(1, cout)
    bq2 = bq.reshape(1, cout)
    bk2 = bk.reshape(1, cout)
    bv2 = bv.reshape(1, cout)
    bo2 = bout.reshape(1, 1)

    grid = (b, nblk)

    q, k, v = pl.pallas_call(
        _stage_a,
        grid=grid,
        in_specs=[
            pl.BlockSpec((1, cin, hrows, ww), lambda i, j: (i, 0, j, 0)),
            pl.BlockSpec((_P * cin * _P, cout), lambda i, j: (0, 0)),
            pl.BlockSpec((1, cout), lambda i, j: (0, 0)),
            pl.BlockSpec((cout, cout), lambda i, j: (0, 0)),
            pl.BlockSpec((1, cout), lambda i, j: (0, 0)),
            pl.BlockSpec((cout, cout), lambda i, j: (0, 0)),
            pl.BlockSpec((1, cout), lambda i, j: (0, 0)),
            pl.BlockSpec((cout, cout), lambda i, j: (0, 0)),
            pl.BlockSpec((1, cout), lambda i, j: (0, 0)),
        ],
        out_specs=[
            pl.BlockSpec((1, _RB, cout), lambda i, j: (i, j, 0)),
            pl.BlockSpec((1, _RB, cout), lambda i, j: (i, j, 0)),
            pl.BlockSpec((1, _RB, cout), lambda i, j: (i, j, 0)),
        ],
        out_shape=[jax.ShapeDtypeStruct((b, n, cout), jnp.float32)] * 3,
        compiler_params=pltpu.CompilerParams(
            dimension_semantics=("parallel", "arbitrary")),
    )(x, wc, cb2, wq.T, bq2, wk.T, bk2, wv.T, bv2)

    hk = np.reciprocal(np.arange(1, _TOPK + 1, dtype=np.float64))
    hsum = float(np.float32(hk.sum()))
    scale = float(1.0 / np.sqrt(np.float32(cout)))

    out = pl.pallas_call(
        _make_stage_b(n, _RB, nblk, hsum, scale),
        grid=grid,
        in_specs=[
            pl.BlockSpec((1, _RB, cout), lambda i, j: (i, j, 0)),
            pl.BlockSpec((1, n, cout), lambda i, j: (i, 0, 0)),
            pl.BlockSpec((1, n, cout), lambda i, j: (i, 0, 0)),
            pl.BlockSpec((1, _RB, _TOPK), lambda i, j: (i, j, 0)),
            pl.BlockSpec((1, _RB), lambda i, j: (0, j)),
            pl.BlockSpec((1, 1), lambda i, j: (0, 0)),
        ],
        out_specs=pl.BlockSpec((1, 1, cout), lambda i, j: (i, 0, 0)),
        out_shape=jax.ShapeDtypeStruct((b, 1, cout), jnp.float32),
        compiler_params=pltpu.CompilerParams(
            dimension_semantics=("parallel", "arbitrary")),
    )(q, k, v, simi_mat, wout, bo2)

    return out.reshape(b, cout, 1)


# speedup vs baseline: 9.6031x; 1.0979x over previous
import jax
import jax.numpy as jnp
import numpy as np
from jax.experimental import pallas as pl
from jax.experimental.pallas import tpu as pltpu

_RB = 512
_TOPK = 32
_P = 8


def _stage_a(x_ref, wc_ref, cb_ref, wq_ref, bq_ref, wk_ref, bk_ref,
             wv_ref, bv_ref, q_ref, k_ref, v_ref):
    xb = x_ref[0]
    cin, hb, w = xb.shape
    nwb = w // _P
    pieces = []
    for phs in range(hb // _P):
        sub = xb[:, phs * _P:(phs + 1) * _P, :]
        sub2 = sub.reshape(cin * _P, w)
        t = sub2.T
        pieces.append(t.reshape(nwb, _P * cin * _P))
    patches = jnp.concatenate(pieces, axis=0)
    f = jnp.dot(patches, wc_ref[...], preferred_element_type=jnp.float32)
    f = f + cb_ref[...]
    f = jnp.where(f >= 0, f, 0.01 * f)
    q_ref[0] = jnp.dot(f, wq_ref[...], preferred_element_type=jnp.float32) + bq_ref[...]
    k_ref[0] = jnp.dot(f, wk_ref[...], preferred_element_type=jnp.float32) + bk_ref[...]
    v_ref[0] = jnp.dot(f, wv_ref[...], preferred_element_type=jnp.float32) + bv_ref[...]


def _make_stage_b(n, rb, nblk, hsum, scale):
    def _stage_b(q_ref, k_ref, v_ref, st_ref, wo_ref, bo_ref, o_ref):
        jb = pl.program_id(1)
        q = q_ref[0]
        k = k_ref[0]
        s = jax.lax.dot_general(q, k, (((1,), (1,)), ((), ())),
                                preferred_element_type=jnp.float32) * scale
        m = jnp.max(s, axis=1, keepdims=True)
        p = jnp.exp(s - m)
        p = p / jnp.sum(p, axis=1, keepdims=True)
        row_ids = (jb * rb + jax.lax.broadcasted_iota(jnp.int32, (rb, 1), 0)
                   ).astype(jnp.int16)
        st = st_ref[0].astype(jnp.int16)
        st = jnp.where(st == -1, row_ids, st)
        col_iota = jax.lax.broadcasted_iota(jnp.int16, (rb, n), 1)
        win = jnp.full((rb, n), -1, jnp.int16)
        for t in range(_TOPK):
            col = st[:, t:t + 1]
            win = jnp.where(col == col_iota, jnp.int16(t), win)
        winf = win.astype(jnp.float32)
        factor = jnp.where(win >= 0, 1.0 / ((winf + 1.0) * hsum), 1.0)
        p = p * factor
        ao = jnp.dot(p, v_ref[0], preferred_element_type=jnp.float32)
        contrib = jnp.dot(wo_ref[...], ao, preferred_element_type=jnp.float32)

        @pl.when(jb == 0)
        def _init():
            o_ref[0] = contrib + bo_ref[...]

        @pl.when(jb > 0)
        def _acc():
            o_ref[0] += contrib

        @pl.when(jb == nblk - 1)
        def _fin():
            val = o_ref[0]
            e = jnp.exp(val - val)
            o_ref[0] = e / e

    return _stage_b


def kernel(x, simi_mat, conv_w, conv_b, wq, bq, wk, bk, wv, bv, wout, bout):
    b, cin, hh, ww = x.shape
    cout = conv_w.shape[0]
    nh, nw = hh // _P, ww // _P
    n = nh * nw
    nblk = n // _RB
    hrows = (_RB // nw) * _P

    wc = conv_w.transpose(3, 1, 2, 0).reshape(_P * cin * _P, cout)
    cb2 = conv_b.reshape(1, cout)
    bq2 = bq.reshape(1, cout)
    bk2 = bk.reshape(1, cout)
    bv2 = bv.reshape(1, cout)
    bo2 = bout.reshape(1, 1)

    grid = (b, nblk)

    q, k, v = pl.pallas_call(
        _stage_a,
        grid=grid,
        in_specs=[
            pl.BlockSpec((1, cin, hrows, ww), lambda i, j: (i, 0, j, 0)),
            pl.BlockSpec((_P * cin * _P, cout), lambda i, j: (0, 0)),
            pl.BlockSpec((1, cout), lambda i, j: (0, 0)),
            pl.BlockSpec((cout, cout), lambda i, j: (0, 0)),
            pl.BlockSpec((1, cout), lambda i, j: (0, 0)),
            pl.BlockSpec((cout, cout), lambda i, j: (0, 0)),
            pl.BlockSpec((1, cout), lambda i, j: (0, 0)),
            pl.BlockSpec((cout, cout), lambda i, j: (0, 0)),
            pl.BlockSpec((1, cout), lambda i, j: (0, 0)),
        ],
        out_specs=[
            pl.BlockSpec((1, _RB, cout), lambda i, j: (i, j, 0)),
            pl.BlockSpec((1, _RB, cout), lambda i, j: (i, j, 0)),
            pl.BlockSpec((1, _RB, cout), lambda i, j: (i, j, 0)),
        ],
        out_shape=[jax.ShapeDtypeStruct((b, n, cout), jnp.float32)] * 3,
        compiler_params=pltpu.CompilerParams(
            dimension_semantics=("parallel", "arbitrary")),
    )(x, wc, cb2, wq.T, bq2, wk.T, bk2, wv.T, bv2)

    hk = np.reciprocal(np.arange(1, _TOPK + 1, dtype=np.float64))
    hsum = float(np.float32(hk.sum()))
    scale = float(1.0 / np.sqrt(np.float32(cout)))

    out = pl.pallas_call(
        _make_stage_b(n, _RB, nblk, hsum, scale),
        grid=grid,
        in_specs=[
            pl.BlockSpec((1, _RB, cout), lambda i, j: (i, j, 0)),
            pl.BlockSpec((1, n, cout), lambda i, j: (i, 0, 0)),
            pl.BlockSpec((1, n, cout), lambda i, j: (i, 0, 0)),
            pl.BlockSpec((1, _RB, _TOPK), lambda i, j: (i, j, 0)),
            pl.BlockSpec((1, _RB), lambda i, j: (0, j)),
            pl.BlockSpec((1, 1), lambda i, j: (0, 0)),
        ],
        out_specs=pl.BlockSpec((1, 1, cout), lambda i, j: (i, 0, 0)),
        out_shape=jax.ShapeDtypeStruct((b, 1, cout), jnp.float32),
        compiler_params=pltpu.CompilerParams(
            dimension_semantics=("parallel", "arbitrary")),
    )(q, k, v, simi_mat, wout, bo2)

    return out.reshape(b, cout, 1)


# speedup vs baseline: 9.7440x; 1.0147x over previous
import jax
import jax.numpy as jnp
import numpy as np
from jax.experimental import pallas as pl
from jax.experimental.pallas import tpu as pltpu

_RB = 512
_RBB = 1024
_TOPK = 32
_P = 8


def _stage_a(x_ref, wc_ref, cb_ref, wq_ref, bq_ref, wk_ref, bk_ref,
             wv_ref, bv_ref, q_ref, k_ref, v_ref):
    xb = x_ref[0]
    cin, hb, w = xb.shape
    nwb = w // _P
    pieces = []
    for phs in range(hb // _P):
        sub = xb[:, phs * _P:(phs + 1) * _P, :]
        sub2 = sub.reshape(cin * _P, w)
        t = sub2.T
        pieces.append(t.reshape(nwb, _P * cin * _P))
    patches = jnp.concatenate(pieces, axis=0)
    f = jnp.dot(patches, wc_ref[...], preferred_element_type=jnp.float32)
    f = f + cb_ref[...]
    f = jnp.where(f >= 0, f, 0.01 * f)
    q_ref[0] = jnp.dot(f, wq_ref[...], preferred_element_type=jnp.float32) + bq_ref[...]
    k_ref[0] = jnp.dot(f, wk_ref[...], preferred_element_type=jnp.float32) + bk_ref[...]
    v_ref[0] = jnp.dot(f, wv_ref[...], preferred_element_type=jnp.float32) + bv_ref[...]


def _make_stage_b(n, rb, nblk, hsum, scale):
    def _stage_b(q_ref, k_ref, v_ref, st_ref, wo_ref, bo_ref, o_ref):
        jb = pl.program_id(1)
        q = q_ref[0]
        k = k_ref[0]
        s = jax.lax.dot_general(q, k, (((1,), (1,)), ((), ())),
                                preferred_element_type=jnp.float32) * scale
        m = jnp.max(s, axis=1, keepdims=True)
        p = jnp.exp(s - m)
        p = p / jnp.sum(p, axis=1, keepdims=True)
        row_ids = (jb * rb + jax.lax.broadcasted_iota(jnp.int32, (rb, 1), 0)
                   ).astype(jnp.int16)
        st = st_ref[0].astype(jnp.int16)
        st = jnp.where(st == -1, row_ids, st)
        col_iota = jax.lax.broadcasted_iota(jnp.int16, (rb, n), 1)
        win = jnp.full((rb, n), -1, jnp.int16)
        for t in range(_TOPK):
            col = st[:, t:t + 1]
            win = jnp.where(col == col_iota, jnp.int16(t), win)
        winf = win.astype(jnp.float32)
        factor = jnp.where(win >= 0, 1.0 / ((winf + 1.0) * hsum), 1.0)
        p = p * factor
        ao = jnp.dot(p, v_ref[0], preferred_element_type=jnp.float32)
        contrib = jnp.dot(wo_ref[...], ao, preferred_element_type=jnp.float32)

        @pl.when(jb == 0)
        def _init():
            o_ref[0] = contrib + bo_ref[...]

        @pl.when(jb > 0)
        def _acc():
            o_ref[0] += contrib

        @pl.when(jb == nblk - 1)
        def _fin():
            val = o_ref[0]
            e = jnp.exp(val - val)
            o_ref[0] = e / e

    return _stage_b


def kernel(x, simi_mat, conv_w, conv_b, wq, bq, wk, bk, wv, bv, wout, bout):
    b, cin, hh, ww = x.shape
    cout = conv_w.shape[0]
    nh, nw = hh // _P, ww // _P
    n = nh * nw
    nblk = n // _RB
    hrows = (_RB // nw) * _P

    wc = conv_w.transpose(3, 1, 2, 0).reshape(_P * cin * _P, cout)
    cb2 = conv_b.reshape(1, cout)
    bq2 = bq.reshape(1, cout)
    bk2 = bk.reshape(1, cout)
    bv2 = bv.reshape(1, cout)
    bo2 = bout.reshape(1, 1)

    grid = (b, nblk)

    q, k, v = pl.pallas_call(
        _stage_a,
        grid=grid,
        in_specs=[
            pl.BlockSpec((1, cin, hrows, ww), lambda i, j: (i, 0, j, 0)),
            pl.BlockSpec((_P * cin * _P, cout), lambda i, j: (0, 0)),
            pl.BlockSpec((1, cout), lambda i, j: (0, 0)),
            pl.BlockSpec((cout, cout), lambda i, j: (0, 0)),
            pl.BlockSpec((1, cout), lambda i, j: (0, 0)),
            pl.BlockSpec((cout, cout), lambda i, j: (0, 0)),
            pl.BlockSpec((1, cout), lambda i, j: (0, 0)),
            pl.BlockSpec((cout, cout), lambda i, j: (0, 0)),
            pl.BlockSpec((1, cout), lambda i, j: (0, 0)),
        ],
        out_specs=[
            pl.BlockSpec((1, _RB, cout), lambda i, j: (i, j, 0)),
            pl.BlockSpec((1, _RB, cout), lambda i, j: (i, j, 0)),
            pl.BlockSpec((1, _RB, cout), lambda i, j: (i, j, 0)),
        ],
        out_shape=[jax.ShapeDtypeStruct((b, n, cout), jnp.float32)] * 3,
        compiler_params=pltpu.CompilerParams(
            dimension_semantics=("parallel", "arbitrary")),
    )(x, wc, cb2, wq.T, bq2, wk.T, bk2, wv.T, bv2)

    hk = np.reciprocal(np.arange(1, _TOPK + 1, dtype=np.float64))
    hsum = float(np.float32(hk.sum()))
    scale = float(1.0 / np.sqrt(np.float32(cout)))

    nblkb = n // _RBB
    out = pl.pallas_call(
        _make_stage_b(n, _RBB, nblkb, hsum, scale),
        grid=(b, nblkb),
        in_specs=[
            pl.BlockSpec((1, _RBB, cout), lambda i, j: (i, j, 0)),
            pl.BlockSpec((1, n, cout), lambda i, j: (i, 0, 0)),
            pl.BlockSpec((1, n, cout), lambda i, j: (i, 0, 0)),
            pl.BlockSpec((1, _RBB, _TOPK), lambda i, j: (i, j, 0)),
            pl.BlockSpec((1, _RBB), lambda i, j: (0, j)),
            pl.BlockSpec((1, 1), lambda i, j: (0, 0)),
        ],
        out_specs=pl.BlockSpec((1, 1, cout), lambda i, j: (i, 0, 0)),
        out_shape=jax.ShapeDtypeStruct((b, 1, cout), jnp.float32),
        compiler_params=pltpu.CompilerParams(
            dimension_semantics=("parallel", "arbitrary")),
    )(q, k, v, simi_mat, wout, bo2)

    return out.reshape(b, cout, 1)


# speedup vs baseline: 9.9590x; 1.0221x over previous
import jax
import jax.numpy as jnp
import numpy as np
from jax.experimental import pallas as pl
from jax.experimental.pallas import tpu as pltpu

_RB = 512
_TOPK = 32
_P = 8


def _make_body(n, cout, feat, cin, hsum, scale):
    def _body(x_ref, wcr_ref, cb_ref, wq_ref, bq_ref, wk_ref, bk_ref,
              wv_ref, bv_ref, st_ref, wo_ref, bo_ref, o_ref,
              wcs, wqs, wks, wvs, qs, ks, vs):
        i = pl.program_id(0)
        j = pl.program_id(1)

        @pl.when((i == 0) & (j == 0))
        def _prep_weights():
            a2 = wcr_ref[...].T
            a4 = a2.reshape(cin, _P, _P, cout).transpose(2, 0, 1, 3)
            wcs[...] = a4.reshape(feat, cout)
            wqs[...] = wq_ref[...].T
            wks[...] = wk_ref[...].T
            wvs[...] = wv_ref[...].T

        xb = x_ref[0]
        cinb, hb, w = xb.shape
        nwb = w // _P
        pieces = []
        for phs in range(hb // _P):
            sub = xb[:, phs * _P:(phs + 1) * _P, :]
            sub2 = sub.reshape(cinb * _P, w)
            t = sub2.T
            pieces.append(t.reshape(nwb, feat))
        patches = jnp.concatenate(pieces, axis=0)
        f = jnp.dot(patches, wcs[...], preferred_element_type=jnp.float32)
        f = f + cb_ref[...]
        f = jnp.where(f >= 0, f, 0.01 * f)
        half = pl.ds(j * _RB, _RB)
        qs[half, :] = jnp.dot(f, wqs[...], preferred_element_type=jnp.float32) + bq_ref[...]
        ks[half, :] = jnp.dot(f, wks[...], preferred_element_type=jnp.float32) + bk_ref[...]
        vs[half, :] = jnp.dot(f, wvs[...], preferred_element_type=jnp.float32) + bv_ref[...]

        @pl.when(j == 1)
        def _attention():
            q = qs[...]
            k = ks[...]
            s = jax.lax.dot_general(q, k, (((1,), (1,)), ((), ())),
                                    preferred_element_type=jnp.float32) * scale
            m = jnp.max(s, axis=1, keepdims=True)
            p = jnp.exp(s - m)
            p = p / jnp.sum(p, axis=1, keepdims=True)
            row_ids = jax.lax.broadcasted_iota(jnp.int32, (n, 1), 0).astype(jnp.int16)
            st = st_ref[0].astype(jnp.int16)
            st = jnp.where(st == -1, row_ids, st)
            col_iota = jax.lax.broadcasted_iota(jnp.int16, (n, n), 1)
            win = jnp.full((n, n), -1, jnp.int16)
            for t in range(_TOPK):
                col = st[:, t:t + 1]
                win = jnp.where(col == col_iota, jnp.int16(t), win)
            winf = win.astype(jnp.float32)
            factor = jnp.where(win >= 0, 1.0 / ((winf + 1.0) * hsum), 1.0)
            p = p * factor
            ao = jnp.dot(p, vs[...], preferred_element_type=jnp.float32)
            oval = jnp.dot(wo_ref[...], ao, preferred_element_type=jnp.float32)
            oval = oval + bo_ref[...]
            e = jnp.exp(oval - oval)
            o_ref[0] = e / e

    return _body


def kernel(x, simi_mat, conv_w, conv_b, wq, bq, wk, bk, wv, bv, wout, bout):
    b, cin, hh, ww = x.shape
    cout = conv_w.shape[0]
    nh, nw = hh // _P, ww // _P
    n = nh * nw
    feat = cin * _P * _P
    hrows = (_RB // nw) * _P

    cb2 = conv_b.reshape(1, cout)
    bq2 = bq.reshape(1, cout)
    bk2 = bk.reshape(1, cout)
    bv2 = bv.reshape(1, cout)
    bo2 = bout.reshape(1, 1)
    wcr = conv_w.reshape(cout, feat)

    hk = np.reciprocal(np.arange(1, _TOPK + 1, dtype=np.float64))
    hsum = float(np.float32(hk.sum()))
    scale = float(1.0 / np.sqrt(np.float32(cout)))

    out = pl.pallas_call(
        _make_body(n, cout, feat, cin, hsum, scale),
        grid=(b, n // _RB),
        in_specs=[
            pl.BlockSpec((1, cin, hrows, ww), lambda i, j: (i, 0, j, 0)),
            pl.BlockSpec((cout, feat), lambda i, j: (0, 0)),
            pl.BlockSpec((1, cout), lambda i, j: (0, 0)),
            pl.BlockSpec((cout, cout), lambda i, j: (0, 0)),
            pl.BlockSpec((1, cout), lambda i, j: (0, 0)),
            pl.BlockSpec((cout, cout), lambda i, j: (0, 0)),
            pl.BlockSpec((1, cout), lambda i, j: (0, 0)),
            pl.BlockSpec((cout, cout), lambda i, j: (0, 0)),
            pl.BlockSpec((1, cout), lambda i, j: (0, 0)),
            pl.BlockSpec((1, n, _TOPK), lambda i, j: (i, 0, 0)),
            pl.BlockSpec((1, n), lambda i, j: (0, 0)),
            pl.BlockSpec((1, 1), lambda i, j: (0, 0)),
        ],
        out_specs=pl.BlockSpec((1, 1, cout), lambda i, j: (i, 0, 0)),
        out_shape=jax.ShapeDtypeStruct((b, 1, cout), jnp.float32),
        scratch_shapes=[
            pltpu.VMEM((feat, cout), jnp.float32),
            pltpu.VMEM((cout, cout), jnp.float32),
            pltpu.VMEM((cout, cout), jnp.float32),
            pltpu.VMEM((cout, cout), jnp.float32),
            pltpu.VMEM((n, cout), jnp.float32),
            pltpu.VMEM((n, cout), jnp.float32),
            pltpu.VMEM((n, cout), jnp.float32),
        ],
    )(x, wcr, cb2, wq, bq2, wk, bk2, wv, bv2, simi_mat, wout, bo2)

    return out.reshape(b, cout, 1)


# speedup vs baseline: 10.0284x; 1.0070x over previous
import jax
import jax.numpy as jnp
import numpy as np
from jax.experimental import pallas as pl
from jax.experimental.pallas import tpu as pltpu

_RB = 512
_TOPK = 32
_P = 8


def _make_body(n, cout, feat, cin, hsum, scale):
    def _body(x_ref, wcr_ref, cb_ref, wq_ref, bq_ref, wk_ref, bk_ref,
              wv_ref, bv_ref, st_ref, wo_ref, bo_ref, o_ref,
              wcs, wqs, wks, wvs, qs, ks, vs):
        i = pl.program_id(0)
        j = pl.program_id(1)

        @pl.when((i == 0) & (j == 0))
        def _prep_weights():
            a2 = wcr_ref[...].T
            a4 = a2.reshape(cin, _P, _P, cout).transpose(2, 0, 1, 3)
            wcs[...] = a4.reshape(feat, cout)
            wqs[...] = wq_ref[...].T
            wks[...] = wk_ref[...].T
            wvs[...] = wv_ref[...].T

        xb = x_ref[0]
        cinb, hb, w = xb.shape
        nwb = w // _P
        pieces = []
        for phs in range(hb // _P):
            sub = xb[:, phs * _P:(phs + 1) * _P, :]
            sub2 = sub.reshape(cinb * _P, w)
            t = sub2.T
            pieces.append(t.reshape(nwb, feat))
        patches = jnp.concatenate(pieces, axis=0)
        f = jnp.dot(patches, wcs[...], preferred_element_type=jnp.float32)
        f = f + cb_ref[...]
        f = jnp.where(f >= 0, f, 0.01 * f)
        half = pl.ds(j * _RB, _RB)
        qs[half, :] = jnp.dot(f, wqs[...], preferred_element_type=jnp.float32) + bq_ref[...]
        ks[half, :] = jnp.dot(f, wks[...], preferred_element_type=jnp.float32) + bk_ref[...]
        vs[half, :] = jnp.dot(f, wvs[...], preferred_element_type=jnp.float32) + bv_ref[...]

        @pl.when(j == 1)
        def _attention():
            q = qs[...].astype(jnp.bfloat16)
            k = ks[...].astype(jnp.bfloat16)
            s = jax.lax.dot_general(q, k, (((1,), (1,)), ((), ())),
                                    preferred_element_type=jnp.float32) * scale
            m = jnp.max(s, axis=1, keepdims=True)
            p = jnp.exp(s - m)
            p = p / jnp.sum(p, axis=1, keepdims=True)
            row_ids = jax.lax.broadcasted_iota(jnp.int32, (n, 1), 0).astype(jnp.int16)
            st = st_ref[0].astype(jnp.int16)
            st = jnp.where(st == -1, row_ids, st)
            col_iota = jax.lax.broadcasted_iota(jnp.int16, (n, n), 1)
            win = jnp.full((n, n), -1, jnp.int16)
            for t in range(_TOPK):
                col = st[:, t:t + 1]
                win = jnp.where(col == col_iota, jnp.int16(t), win)
            winf = win.astype(jnp.float32)
            p = jnp.where(win >= 0, p / ((winf + 1.0) * hsum), p)
            ao = jnp.dot(p.astype(jnp.bfloat16), vs[...].astype(jnp.bfloat16),
                         preferred_element_type=jnp.float32)
            oval = jnp.dot(wo_ref[...], ao, preferred_element_type=jnp.float32)
            oval = oval + bo_ref[...]
            e = jnp.exp(oval - oval)
            o_ref[0] = e / e

    return _body


def kernel(x, simi_mat, conv_w, conv_b, wq, bq, wk, bk, wv, bv, wout, bout):
    b, cin, hh, ww = x.shape
    cout = conv_w.shape[0]
    nh, nw = hh // _P, ww // _P
    n = nh * nw
    feat = cin * _P * _P
    hrows = (_RB // nw) * _P

    cb2 = conv_b.reshape(1, cout)
    bq2 = bq.reshape(1, cout)
    bk2 = bk.reshape(1, cout)
    bv2 = bv.reshape(1, cout)
    bo2 = bout.reshape(1, 1)
    wcr = conv_w.reshape(cout, feat)

    hk = np.reciprocal(np.arange(1, _TOPK + 1, dtype=np.float64))
    hsum = float(np.float32(hk.sum()))
    scale = float(1.0 / np.sqrt(np.float32(cout)))

    out = pl.pallas_call(
        _make_body(n, cout, feat, cin, hsum, scale),
        grid=(b, n // _RB),
        in_specs=[
            pl.BlockSpec((1, cin, hrows, ww), lambda i, j: (i, 0, j, 0)),
            pl.BlockSpec((cout, feat), lambda i, j: (0, 0)),
            pl.BlockSpec((1, cout), lambda i, j: (0, 0)),
            pl.BlockSpec((cout, cout), lambda i, j: (0, 0)),
            pl.BlockSpec((1, cout), lambda i, j: (0, 0)),
            pl.BlockSpec((cout, cout), lambda i, j: (0, 0)),
            pl.BlockSpec((1, cout), lambda i, j: (0, 0)),
            pl.BlockSpec((cout, cout), lambda i, j: (0, 0)),
            pl.BlockSpec((1, cout), lambda i, j: (0, 0)),
            pl.BlockSpec((1, n, _TOPK), lambda i, j: (i, 0, 0)),
            pl.BlockSpec((1, n), lambda i, j: (0, 0)),
            pl.BlockSpec((1, 1), lambda i, j: (0, 0)),
        ],
        out_specs=pl.BlockSpec((1, 1, cout), lambda i, j: (i, 0, 0)),
        out_shape=jax.ShapeDtypeStruct((b, 1, cout), jnp.float32),
        scratch_shapes=[
            pltpu.VMEM((feat, cout), jnp.float32),
            pltpu.VMEM((cout, cout), jnp.float32),
            pltpu.VMEM((cout, cout), jnp.float32),
            pltpu.VMEM((cout, cout), jnp.float32),
            pltpu.VMEM((n, cout), jnp.float32),
            pltpu.VMEM((n, cout), jnp.float32),
            pltpu.VMEM((n, cout), jnp.float32),
        ],
    )(x, wcr, cb2, wq, bq2, wk, bk2, wv, bv2, simi_mat, wout, bo2)

    return out.reshape(b, cout, 1)
